# Initial kernel scaffold; baseline (speedup 1.0000x reference)
import jax
import jax.numpy as jnp
from jax.experimental import pallas as pl


def kernel(x_drug, x_disease, x_protein, x_gene, x_pathway, params, edge_drug_drug, edge_drug_disease, edge_disease_disease, edge_drug_protein, edge_protein_protein, edge_protein_gene, edge_gene_gene, edge_gene_pathway, edge_pathway_pathway, edge_pathway_disease):
    raise NotImplementedError("write your pallas kernel here")


# speedup vs baseline: 4.2998x; 4.2998x over previous
import functools

import jax
import jax.numpy as jnp
from jax import lax
from jax.experimental import pallas as pl
from jax.experimental.pallas import tpu as pltpu
from jax.experimental.pallas import tpu_sc as plsc

_N = 10000
_D = 128
_E = 160000
_R = 10
_CH = 128
_NCH = _E // _CH
_SIDE = _R * _E

_RELS = ['drug_drug', 'drug_disease', 'disease_disease', 'drug_protein',
         'protein_protein', 'protein_gene', 'gene_gene', 'gene_pathway',
         'pathway_pathway', 'pathway_disease']
_SRC_T = [0, 0, 1, 0, 2, 2, 3, 3, 4, 4]

_ZBLOCK = ['dd', 'dp', 'dd', 'pd', 'dp', 'pg', 'pg', 'gp', 'gp', 'pd']
_CONTRIB = [
    [(0, 'drug_drug')],
    [(0, 'drug_drug')],
    [(1, 'drug_disease'), (2, 'disease_disease')],
    [(9, 'pathway_disease'), (2, 'disease_disease')],
    [(3, 'drug_protein'), (4, 'protein_protein')],
    [(4, 'protein_protein')],
    [(5, 'protein_gene'), (6, 'gene_gene')],
    [(6, 'gene_gene')],
    [(7, 'gene_pathway'), (8, 'pathway_pathway')],
    [(8, 'pathway_pathway')],
]
_WLIST = []
_CONTRIB_IDX = []
for _zi, _lst in enumerate(_CONTRIB):
    _idxs = []
    for (_r, _rl) in _lst:
        _idxs.append((_r, len(_WLIST)))
        _WLIST.append((_ZBLOCK[_zi], _rl))
    _CONTRIB_IDX.append(_idxs)


def _deg_body(ecat, degp, bins, idxb):
    c = lax.axis_index("c")
    s = lax.axis_index("s")
    zero16 = jnp.zeros((16,), jnp.float32)
    ones16 = jnp.ones((16,), jnp.float32)

    @pl.loop(0, _R * _N // 16)
    def _zero(j):
        bins[pl.ds(j * 16, 16)] = zero16

    nch = 2 * _R * _NCH // 2

    @pl.loop(0, (nch + 15) // 16)
    def _chunks(t):
        ch = s + 16 * t

        @pl.when(ch < nch)
        def _():
            off = c * _SIDE + ch * _CH
            pltpu.sync_copy(ecat.at[pl.ds(off, _CH)], idxb)
            for j in range(8):
                v = idxb[pl.ds(j * 16, 16)]
                plsc.addupdate_scatter(bins, [v], ones16)

    w = c * 16 + s
    for r in range(_R):
        pltpu.sync_copy(bins.at[pl.ds(r * _N, _N)],
                        degp.at[pl.ds((r * 32 + w) * _N, _N)])


def _run_deg(ecat):
    mesh = plsc.VectorSubcoreMesh(core_axis_name="c", subcore_axis_name="s",
                                  num_cores=2, num_subcores=16)
    return pl.kernel(
        _deg_body,
        out_type=jax.ShapeDtypeStruct((32 * _R * _N,), jnp.float32),
        mesh=mesh,
        compiler_params=pltpu.CompilerParams(needs_layout_passes=False),
        scratch_types=[
            pltpu.VMEM((_R * _N,), jnp.float32),
            pltpu.VMEM((_CH,), jnp.int32),
        ],
    )(ecat)


def _norm_body(x_ref, deg_ref, xs_ref, inn_ref):
    deg = deg_ref[...]
    outd = jnp.sum(deg[0, 0, :, :], axis=0)
    ind = jnp.sum(deg[0, 1, :, :], axis=0)
    onorm = lax.rsqrt(jnp.maximum(outd, 1.0))
    inorm = lax.rsqrt(jnp.maximum(ind, 1.0))
    xs_ref[0] = x_ref[0] * onorm[:, None]
    inn_ref[0, 0] = inorm


def _src_t_of(r):
    return jnp.where(r >= 4, r // 2, jnp.where(r == 2, 1, 0))


def _run_norm(x5, deg4):
    return pl.pallas_call(
        _norm_body,
        grid=(_R,),
        in_specs=[
            pl.BlockSpec((1, _N, _D), lambda r: (_src_t_of(r), 0, 0)),
            pl.BlockSpec((1, 2, 16, _N), lambda r: (r, 0, 0, 0)),
        ],
        out_specs=[
            pl.BlockSpec((1, _N, _D), lambda r: (r, 0, 0)),
            pl.BlockSpec((1, 1, _N), lambda r: (r, 0, 0)),
        ],
        out_shape=[
            jax.ShapeDtypeStruct((_R, _N, _D), jnp.float32),
            jax.ShapeDtypeStruct((_R, 1, _N), jnp.float32),
        ],
    )(x5, deg4)


def _agg_body(xsf, srcf, dstf, agg, acc, idx2, rows, zrow, sem):
    c = lax.axis_index("c")
    s = lax.axis_index("s")
    zero16 = jnp.zeros((16,), jnp.float32)

    @pl.loop(0, 80)
    def _zz(j):
        for t in range(8):
            zrow[j, pl.ds(t * 16, 16)] = zero16

    def _zero_stripe():
        @pl.loop(0, 8)
        def _zs(t):
            g = s + 16 * t

            @pl.when(g < 125)
            def _():
                pltpu.sync_copy(zrow, acc.at[pl.ds(g * 80, 80)])

    _zero_stripe()
    for i in range(_R // 2):
        r = i * 2 + c
        plsc.subcore_barrier()

        @pl.loop(0, (_NCH + 15) // 16)
        def _chunks(t):
            ch = s + 16 * t

            @pl.when(ch < _NCH)
            def _():
                off = r * _E + ch * _CH
                pltpu.sync_copy(srcf.at[pl.ds(off, _CH)], idx2.at[0])
                pltpu.sync_copy(dstf.at[pl.ds(off, _CH)], idx2.at[1])
                pltpu.async_copy(xsf.at[idx2.at[0]], rows, sem).wait()
                pltpu.sync_copy(rows, acc.at[idx2.at[1]], add=True)

        plsc.subcore_barrier()

        @pl.loop(0, 8)
        def _drain(t):
            g = s + 16 * t

            @pl.when(g < 125)
            def _():
                sl = pl.ds(g * 80, 80)
                pltpu.sync_copy(acc.at[sl], agg.at[r, sl])

        if i < _R // 2 - 1:
            _zero_stripe()


def _run_agg(xsf, srcf, dstf):
    mesh = plsc.VectorSubcoreMesh(core_axis_name="c", subcore_axis_name="s",
                                  num_cores=2, num_subcores=16)
    return pl.kernel(
        _agg_body,
        out_type=jax.ShapeDtypeStruct((_R, _N, _D), jnp.float32),
        mesh=mesh,
        compiler_params=pltpu.CompilerParams(needs_layout_passes=False),
        scratch_types=[
            pltpu.VMEM_SHARED((_N, _D), jnp.float32),
            pltpu.VMEM((2, _CH), jnp.int32),
            pltpu.VMEM((_CH, _D), jnp.float32),
            pltpu.VMEM((80, _D), jnp.float32),
            pltpu.SemaphoreType.DMA,
        ],
    )(xsf, srcf, dstf)


_TB = 1000


def _blk_body(agg_ref, inn_ref, ws_ref, bsum_ref, at_ref, w1_ref, b1_ref,
              w2_ref, z_ref, s2_ref):
    g = pl.program_id(0)
    w1 = w1_ref[...]
    b1 = b1_ref[...]
    w2r = w2_ref[...]
    sc = [agg_ref[r] * inn_ref[:, r][:, None] for r in range(_R)]
    for zi in range(10):
        tot = None
        for (r, wi) in _CONTRIB_IDX[zi]:
            y = jnp.dot(sc[r], ws_ref[wi], preferred_element_type=jnp.float32)
            tot = y if tot is None else tot + y
        tot = tot + bsum_ref[zi]
        z = jnp.where(tot >= 0, tot, tot * at_ref[zi])
        z_ref[zi] = z
        t = jnp.tanh(jnp.dot(z, w1, preferred_element_type=jnp.float32) + b1)
        sco = jnp.sum(t * w2r)
        srow = jnp.full((_D,), sco, jnp.float32)

        @pl.when(g == 0)
        def _():
            s2_ref[zi] = srow

        @pl.when(g > 0)
        def _():
            s2_ref[zi] = s2_ref[zi] + srow


def _run_blk(agg, inn, ws, bsum, at, w1, b1r, w2r):
    return pl.pallas_call(
        _blk_body,
        grid=(_N // _TB,),
        in_specs=[
            pl.BlockSpec((_R, _TB, _D), lambda g: (0, g, 0)),
            pl.BlockSpec((_TB, _R), lambda g: (g, 0)),
            pl.BlockSpec((15, _D, _D), lambda g: (0, 0, 0)),
            pl.BlockSpec((10, 1, _D), lambda g: (0, 0, 0)),
            pl.BlockSpec((10, 1, _D), lambda g: (0, 0, 0)),
            pl.BlockSpec((_D, _D), lambda g: (0, 0)),
            pl.BlockSpec((1, _D), lambda g: (0, 0)),
            pl.BlockSpec((1, _D), lambda g: (0, 0)),
        ],
        out_specs=[
            pl.BlockSpec((10, _TB, _D), lambda g: (0, g, 0)),
            pl.BlockSpec((10, _D), lambda g: (0, 0)),
        ],
        out_shape=[
            jax.ShapeDtypeStruct((10, _N, _D), jnp.float32),
            jax.ShapeDtypeStruct((10, _D), jnp.float32),
        ],
    )(agg, inn, ws, bsum, at, w1, b1r, w2r)


def _comb_body(z_ref, s2_ref, o0, o1, o2, o3, o4):
    outs = [o0, o1, o2, o3, o4]
    for nt in range(5):
        s0 = s2_ref[2 * nt]
        s1 = s2_ref[2 * nt + 1]
        m = jnp.maximum(s0, s1)
        e0 = jnp.exp((s0 - m) * (1.0 / _N))
        e1 = jnp.exp((s1 - m) * (1.0 / _N))
        b0 = e0 / (e0 + e1)
        b1 = e1 / (e0 + e1)
        outs[nt][...] = (z_ref[2 * nt] * b0[None, :]
                         + z_ref[2 * nt + 1] * b1[None, :])


def _run_comb(z, s2):
    return pl.pallas_call(
        _comb_body,
        grid=(_N // _TB,),
        in_specs=[
            pl.BlockSpec((10, _TB, _D), lambda g: (0, g, 0)),
            pl.BlockSpec((10, _D), lambda g: (0, 0)),
        ],
        out_specs=[pl.BlockSpec((_TB, _D), lambda g: (g, 0))] * 5,
        out_shape=[jax.ShapeDtypeStruct((_N, _D), jnp.float32)] * 5,
    )(z, s2)


def kernel(x_drug, x_disease, x_protein, x_gene, x_pathway, params,
           edge_drug_drug, edge_drug_disease, edge_disease_disease,
           edge_drug_protein, edge_protein_protein, edge_protein_gene,
           edge_gene_gene, edge_gene_pathway, edge_pathway_pathway,
           edge_pathway_disease):
    edges = [edge_drug_drug, edge_drug_disease, edge_disease_disease,
             edge_drug_protein, edge_protein_protein, edge_protein_gene,
             edge_gene_gene, edge_gene_pathway, edge_pathway_pathway,
             edge_pathway_disease]
    x5 = jnp.stack([x_drug, x_disease, x_protein, x_gene, x_pathway])

    offs = (jnp.arange(_R, dtype=jnp.int32) * _N)[:, None]
    srcs = jnp.stack([e[0] for e in edges])
    dsts = jnp.stack([e[1] for e in edges])
    srca = (srcs + offs).reshape(-1)
    dsta = (dsts + offs).reshape(-1)
    ecat = jnp.concatenate([srca, dsta])
    dstf = dsts.reshape(-1)

    degp = _run_deg(ecat)
    deg4 = degp.reshape(_R, 2, 16, _N)
    xs, inn = _run_norm(x5, deg4)
    innt = inn[:, 0, :].T
    xsf = xs.reshape(_R * _N, _D)
    agg = _run_agg(xsf, srca, dstf)

    ws = jnp.stack([params['W_%s_%s' % (b, rl)] for (b, rl) in _WLIST])
    bsum = jnp.stack([
        sum(params['b_%s_%s' % (_ZBLOCK[zi], rl)] for (_, rl) in _CONTRIB[zi])
        for zi in range(10)]).reshape(10, 1, _D)
    at = jnp.stack([jnp.full((_D,), params['a_' + blk]) for blk in _ZBLOCK]
                   ).reshape(10, 1, _D)
    b1r = params['att_b1'].reshape(1, _D)
    w2r = params['att_w2'].reshape(1, _D)

    z, s2 = _run_blk(agg, innt, ws, bsum, at, params['att_W1'], b1r, w2r)
    outs = _run_comb(z, s2)
    return tuple(outs)


# speedup vs baseline: 8.3738x; 1.9475x over previous
import functools

import jax
import jax.numpy as jnp
from jax import lax
from jax.experimental import pallas as pl
from jax.experimental.pallas import tpu as pltpu
from jax.experimental.pallas import tpu_sc as plsc

_N = 10000
_D = 128
_E = 160000
_R = 10
_CH = 128
_NCH = _E // _CH
_SIDE = _R * _E

_RELS = ['drug_drug', 'drug_disease', 'disease_disease', 'drug_protein',
         'protein_protein', 'protein_gene', 'gene_gene', 'gene_pathway',
         'pathway_pathway', 'pathway_disease']
_SRC_T = [0, 0, 1, 0, 2, 2, 3, 3, 4, 4]

_ZBLOCK = ['dd', 'dp', 'dd', 'pd', 'dp', 'pg', 'pg', 'gp', 'gp', 'pd']
_CONTRIB = [
    [(0, 'drug_drug')],
    [(0, 'drug_drug')],
    [(1, 'drug_disease'), (2, 'disease_disease')],
    [(9, 'pathway_disease'), (2, 'disease_disease')],
    [(3, 'drug_protein'), (4, 'protein_protein')],
    [(4, 'protein_protein')],
    [(5, 'protein_gene'), (6, 'gene_gene')],
    [(6, 'gene_gene')],
    [(7, 'gene_pathway'), (8, 'pathway_pathway')],
    [(8, 'pathway_pathway')],
]
_WLIST = []
_CONTRIB_IDX = []
for _zi, _lst in enumerate(_CONTRIB):
    _idxs = []
    for (_r, _rl) in _lst:
        _idxs.append((_r, len(_WLIST)))
        _WLIST.append((_ZBLOCK[_zi], _rl))
    _CONTRIB_IDX.append(_idxs)


_CE = 2000


def _deg_body(ecat, degp, bins, idxb0, idxb1, sem_i0, sem_i1):
    idxb = [idxb0, idxb1]
    c = lax.axis_index("c")
    s = lax.axis_index("s")
    zero16 = jnp.zeros((16,), jnp.float32)
    ones16 = jnp.ones((16,), jnp.float32)
    sems = [sem_i0, sem_i1]

    @pl.loop(0, _R * _N // 16)
    def _zero(j):
        bins[pl.ds(j * 16, 16)] = zero16

    nch = _SIDE // _CE // 16

    def _issue(t, b):
        off = c * _SIDE + (s + 16 * t) * _CE
        pltpu.async_copy(ecat.at[pl.ds(off, _CE)], idxb[b], sems[b])

    _issue(0, 0)
    _issue(1, 1)

    @pl.loop(0, nch // 2)
    def _chunks(g):
        for b in range(2):
            t = g * 2 + b
            pltpu.make_async_copy(ecat.at[pl.ds(0, _CE)], idxb[b],
                                  sems[b]).wait()

            @pl.loop(0, _CE // 16, unroll=4)
            def _scat(j):
                v = idxb[b][pl.ds(j * 16, 16)]
                plsc.addupdate_scatter(bins, [v], ones16)

            @pl.when(t + 2 < nch)
            def _():
                _issue(t + 2, b)

    w = c * 16 + s
    for r in range(_R):
        pltpu.sync_copy(bins.at[pl.ds(r * _N, _N)],
                        degp.at[pl.ds((r * 32 + w) * _N, _N)])


def _run_deg(ecat):
    mesh = plsc.VectorSubcoreMesh(core_axis_name="c", subcore_axis_name="s",
                                  num_cores=2, num_subcores=16)
    return pl.kernel(
        _deg_body,
        out_type=jax.ShapeDtypeStruct((32 * _R * _N,), jnp.float32),
        mesh=mesh,
        compiler_params=pltpu.CompilerParams(needs_layout_passes=False),
        scratch_types=[
            pltpu.VMEM((_R * _N,), jnp.float32),
            pltpu.VMEM((_CE,), jnp.int32),
            pltpu.VMEM((_CE,), jnp.int32),
            pltpu.SemaphoreType.DMA,
            pltpu.SemaphoreType.DMA,
        ],
    )(ecat)


def _norm_body(x_ref, deg_ref, xs_ref, inn_ref):
    deg = deg_ref[...]
    outd = jnp.sum(deg[0, 0, :, :], axis=0)
    ind = jnp.sum(deg[0, 1, :, :], axis=0)
    onorm = lax.rsqrt(jnp.maximum(outd, 1.0))
    inorm = lax.rsqrt(jnp.maximum(ind, 1.0))
    xs_ref[0] = x_ref[0] * onorm[:, None]
    inn_ref[0, 0] = inorm


def _src_t_of(r):
    return jnp.where(r >= 4, r // 2, jnp.where(r == 2, 1, 0))


def _run_norm(x5, deg4):
    return pl.pallas_call(
        _norm_body,
        grid=(_R,),
        in_specs=[
            pl.BlockSpec((1, _N, _D), lambda r: (_src_t_of(r), 0, 0)),
            pl.BlockSpec((1, 2, 16, _N), lambda r: (r, 0, 0, 0)),
        ],
        out_specs=[
            pl.BlockSpec((1, _N, _D), lambda r: (r, 0, 0)),
            pl.BlockSpec((1, 1, _N), lambda r: (r, 0, 0)),
        ],
        out_shape=[
            jax.ShapeDtypeStruct((_R, _N, _D), jnp.float32),
            jax.ShapeDtypeStruct((_R, 1, _N), jnp.float32),
        ],
    )(x5, deg4)


def _agg_body(xsf, srcf, dstf, agg, acc, idx2, rows, zrow,
              sem_i0, sem_i1, sem_g0, sem_g1):
    c = lax.axis_index("c")
    s = lax.axis_index("s")
    zero16 = jnp.zeros((16,), jnp.float32)
    sems_i = [sem_i0, sem_i1]
    sems_g = [sem_g0, sem_g1]

    @pl.loop(0, 80)
    def _zz(j):
        for t in range(8):
            zrow[j, pl.ds(t * 16, 16)] = zero16

    def _zero_stripe():
        @pl.loop(0, 8)
        def _zs(t):
            g = s + 16 * t

            @pl.when(g < 125)
            def _():
                pltpu.sync_copy(zrow, acc.at[pl.ds(g * 80, 80)])

    _zero_stripe()
    for i in range(_R // 2):
        r = i * 2 + c
        plsc.subcore_barrier()

        def _issue_idx(t, b):
            @pl.when(s + 16 * t < _NCH)
            def _():
                off = r * _E + (s + 16 * t) * _CH
                pltpu.async_copy(srcf.at[pl.ds(off, _CH)], idx2.at[b, 0],
                                 sems_i[b])
                pltpu.async_copy(dstf.at[pl.ds(off, _CH)], idx2.at[b, 1],
                                 sems_i[b])

        def _wait_idx(t, b):
            @pl.when(s + 16 * t < _NCH)
            def _():
                pltpu.make_async_copy(srcf.at[pl.ds(0, _CH)],
                                      idx2.at[b, 0], sems_i[b]).wait()
                pltpu.make_async_copy(srcf.at[pl.ds(0, _CH)],
                                      idx2.at[b, 1], sems_i[b]).wait()

        def _issue_gather(t, b):
            @pl.when(s + 16 * t < _NCH)
            def _():
                pltpu.async_copy(xsf.at[idx2.at[b, 0]], rows.at[b],
                                 sems_g[b])

        def _wait_gather(t, b):
            @pl.when(s + 16 * t < _NCH)
            def _():
                pltpu.make_async_copy(xsf.at[pl.ds(0, _CH)], rows.at[b],
                                      sems_g[b]).wait()

        def _scatter(t, b):
            @pl.when(s + 16 * t < _NCH)
            def _():
                pltpu.sync_copy(rows.at[b], acc.at[idx2.at[b, 1]], add=True)

        _issue_idx(0, 0)
        _wait_idx(0, 0)
        _issue_gather(0, 0)
        _issue_idx(1, 1)

        @pl.loop(0, 40)
        def _chunks(g):
            for b in range(2):
                t = g * 2 + b
                _wait_idx(t + 1, 1 - b)
                _wait_gather(t, b)
                _issue_gather(t + 1, 1 - b)
                _scatter(t, b)
                _issue_idx(t + 2, b)

        plsc.subcore_barrier()

        @pl.loop(0, 8)
        def _drain(t):
            g = s + 16 * t

            @pl.when(g < 125)
            def _():
                sl = pl.ds(g * 80, 80)
                pltpu.sync_copy(acc.at[sl], agg.at[r, sl])

        if i < _R // 2 - 1:
            _zero_stripe()


def _run_agg(xsf, srcf, dstf):
    mesh = plsc.VectorSubcoreMesh(core_axis_name="c", subcore_axis_name="s",
                                  num_cores=2, num_subcores=16)
    return pl.kernel(
        _agg_body,
        out_type=jax.ShapeDtypeStruct((_R, _N, _D), jnp.float32),
        mesh=mesh,
        compiler_params=pltpu.CompilerParams(needs_layout_passes=False),
        scratch_types=[
            pltpu.VMEM_SHARED((_N, _D), jnp.float32),
            pltpu.VMEM((2, 2, _CH), jnp.int32),
            pltpu.VMEM((2, _CH, _D), jnp.float32),
            pltpu.VMEM((80, _D), jnp.float32),
            pltpu.SemaphoreType.DMA,
            pltpu.SemaphoreType.DMA,
            pltpu.SemaphoreType.DMA,
            pltpu.SemaphoreType.DMA,
        ],
    )(xsf, srcf, dstf)


_TB = 1000


def _blk_body(agg_ref, inn_ref, ws_ref, bsum_ref, at_ref, w1_ref, b1_ref,
              w2_ref, z_ref, s2_ref):
    g = pl.program_id(0)
    w1 = w1_ref[...]
    b1 = b1_ref[...]
    w2r = w2_ref[...]
    sc = [agg_ref[r] * inn_ref[:, r][:, None] for r in range(_R)]
    for zi in range(10):
        tot = None
        for (r, wi) in _CONTRIB_IDX[zi]:
            y = jnp.dot(sc[r], ws_ref[wi], preferred_element_type=jnp.float32)
            tot = y if tot is None else tot + y
        tot = tot + bsum_ref[zi]
        z = jnp.where(tot >= 0, tot, tot * at_ref[zi])
        z_ref[zi] = z
        t = jnp.tanh(jnp.dot(z, w1, preferred_element_type=jnp.float32) + b1)
        sco = jnp.sum(t * w2r)
        srow = jnp.full((_D,), sco, jnp.float32)

        @pl.when(g == 0)
        def _():
            s2_ref[zi] = srow

        @pl.when(g > 0)
        def _():
            s2_ref[zi] = s2_ref[zi] + srow


def _run_blk(agg, inn, ws, bsum, at, w1, b1r, w2r):
    return pl.pallas_call(
        _blk_body,
        grid=(_N // _TB,),
        in_specs=[
            pl.BlockSpec((_R, _TB, _D), lambda g: (0, g, 0)),
            pl.BlockSpec((_TB, _R), lambda g: (g, 0)),
            pl.BlockSpec((15, _D, _D), lambda g: (0, 0, 0)),
            pl.BlockSpec((10, 1, _D), lambda g: (0, 0, 0)),
            pl.BlockSpec((10, 1, _D), lambda g: (0, 0, 0)),
            pl.BlockSpec((_D, _D), lambda g: (0, 0)),
            pl.BlockSpec((1, _D), lambda g: (0, 0)),
            pl.BlockSpec((1, _D), lambda g: (0, 0)),
        ],
        out_specs=[
            pl.BlockSpec((10, _TB, _D), lambda g: (0, g, 0)),
            pl.BlockSpec((10, _D), lambda g: (0, 0)),
        ],
        out_shape=[
            jax.ShapeDtypeStruct((10, _N, _D), jnp.float32),
            jax.ShapeDtypeStruct((10, _D), jnp.float32),
        ],
    )(agg, inn, ws, bsum, at, w1, b1r, w2r)


def _comb_body(z_ref, s2_ref, o0, o1, o2, o3, o4):
    outs = [o0, o1, o2, o3, o4]
    for nt in range(5):
        s0 = s2_ref[2 * nt]
        s1 = s2_ref[2 * nt + 1]
        m = jnp.maximum(s0, s1)
        e0 = jnp.exp((s0 - m) * (1.0 / _N))
        e1 = jnp.exp((s1 - m) * (1.0 / _N))
        b0 = e0 / (e0 + e1)
        b1 = e1 / (e0 + e1)
        outs[nt][...] = (z_ref[2 * nt] * b0[None, :]
                         + z_ref[2 * nt + 1] * b1[None, :])


def _run_comb(z, s2):
    return pl.pallas_call(
        _comb_body,
        grid=(_N // _TB,),
        in_specs=[
            pl.BlockSpec((10, _TB, _D), lambda g: (0, g, 0)),
            pl.BlockSpec((10, _D), lambda g: (0, 0)),
        ],
        out_specs=[pl.BlockSpec((_TB, _D), lambda g: (g, 0))] * 5,
        out_shape=[jax.ShapeDtypeStruct((_N, _D), jnp.float32)] * 5,
    )(z, s2)


def kernel(x_drug, x_disease, x_protein, x_gene, x_pathway, params,
           edge_drug_drug, edge_drug_disease, edge_disease_disease,
           edge_drug_protein, edge_protein_protein, edge_protein_gene,
           edge_gene_gene, edge_gene_pathway, edge_pathway_pathway,
           edge_pathway_disease):
    edges = [edge_drug_drug, edge_drug_disease, edge_disease_disease,
             edge_drug_protein, edge_protein_protein, edge_protein_gene,
             edge_gene_gene, edge_gene_pathway, edge_pathway_pathway,
             edge_pathway_disease]
    x5 = jnp.stack([x_drug, x_disease, x_protein, x_gene, x_pathway])

    offs = (jnp.arange(_R, dtype=jnp.int32) * _N)[:, None]
    srcs = jnp.stack([e[0] for e in edges])
    dsts = jnp.stack([e[1] for e in edges])
    srca = (srcs + offs).reshape(-1)
    dsta = (dsts + offs).reshape(-1)
    ecat = jnp.concatenate([srca, dsta])
    dstf = dsts.reshape(-1)

    degp = _run_deg(ecat)
    deg4 = degp.reshape(_R, 2, 16, _N)
    xs, inn = _run_norm(x5, deg4)
    innt = inn[:, 0, :].T
    xsf = xs.reshape(_R * _N, _D)
    agg = _run_agg(xsf, srca, dstf)

    ws = jnp.stack([params['W_%s_%s' % (b, rl)] for (b, rl) in _WLIST])
    bsum = jnp.stack([
        sum(params['b_%s_%s' % (_ZBLOCK[zi], rl)] for (_, rl) in _CONTRIB[zi])
        for zi in range(10)]).reshape(10, 1, _D)
    at = jnp.stack([jnp.full((_D,), params['a_' + blk]) for blk in _ZBLOCK]
                   ).reshape(10, 1, _D)
    b1r = params['att_b1'].reshape(1, _D)
    w2r = params['att_w2'].reshape(1, _D)

    z, s2 = _run_blk(agg, innt, ws, bsum, at, params['att_W1'], b1r, w2r)
    outs = _run_comb(z, s2)
    return tuple(outs)
